# baseline (speedup 1.0000x reference)
import functools

import jax
import jax.numpy as jnp
import numpy as np
from jax import lax
from jax.experimental import pallas as pl
from jax.experimental.pallas import tpu as pltpu
from jax.experimental.pallas import tpu_sc as plsc

N = 10000
E = 320000
DIM = 128
SW = 128
NC = 2
NS = 16
NW = NC * NS
NP = 10008
K = 48
NCH = 209
EPW = K * NCH
EP = NW * EPW
RPT = 624
TAIL = NP - NS * RPT
DR = 1256

_PERM = np.empty((DIM,), np.int64)
for _c in range(DIM):
  _j, _l = _c >> 4, _c & 15
  _PERM[_c] = (_l & 7) * 16 + 2 * _j + (_l >> 3)
_PINV = np.zeros((DIM, DIM), np.float32)
for _c in range(DIM):
  _PINV[_c, _PERM[_c]] = 1.0


def _edge_pass_build():
  mesh = plsc.VectorSubcoreMesh(core_axis_name="c", subcore_axis_name="s")

  @functools.partial(
      pl.kernel,
      out_type=(jax.ShapeDtypeStruct((NC, NP, DIM), jnp.float32),
                jax.ShapeDtypeStruct((NC, DR, DIM), jnp.float32)),
      mesh=mesh,
      scratch_types=[
          pltpu.VMEM((K,), jnp.int32),
          pltpu.VMEM((K,), jnp.int32),
          pltpu.VMEM((K,), jnp.int32),
          pltpu.VMEM((K, DIM), jnp.float32),
          pltpu.VMEM((K, SW), jnp.float32),
          pltpu.VMEM((K, SW), jnp.float32),
          pltpu.VMEM((K, DIM), jnp.float32),
          pltpu.VMEM((K, DIM), jnp.float32),
          pltpu.VMEM_SHARED((NP, DIM), jnp.float32),
          pltpu.VMEM_SHARED((DR, DIM), jnp.float32),
          pltpu.SemaphoreType.DMA,
      ])
  def edge_pass(zt_hbm, sl_hbm, sr_hbm, edges_hbm, zeros_hbm, u_out, d_out,
                src_v, dst_v, drow_v, zb, ssb, sdb, wb, wd, u_sh, d_sh, sem):
    c = lax.axis_index("c")
    s = lax.axis_index("s")
    wid = s * NC + c

    pltpu.sync_copy(zeros_hbm.at[pl.ds(s * RPT, RPT)],
                    u_sh.at[pl.ds(s * RPT, RPT)])

    @pl.when(s == 0)
    def _():
      pltpu.sync_copy(zeros_hbm.at[pl.ds(NS * RPT, TAIL)],
                      u_sh.at[pl.ds(NS * RPT, TAIL)])
      pltpu.sync_copy(zeros_hbm.at[pl.ds(0, DR)], d_sh)

    def zrow(k, carry):
      for j in range(8):
        wd[k, pl.ds(16 * j, 16)] = jnp.zeros((16,), jnp.float32)
      return carry

    lax.fori_loop(0, K, zrow, 0)
    plsc.subcore_barrier()

    ebase = wid * EPW

    def chunk(ci, carry):
      b = ebase + ci * K
      pltpu.sync_copy(edges_hbm.at[pl.ds(b, K)], src_v)
      pltpu.sync_copy(edges_hbm.at[pl.ds(EP + b, K)], dst_v)
      c1 = pltpu.async_copy(zt_hbm.at[src_v], zb, sem)
      c2 = pltpu.async_copy(sl_hbm.at[src_v], ssb, sem)
      c3 = pltpu.async_copy(sr_hbm.at[dst_v], sdb, sem)
      c1.wait()
      c2.wait()
      c3.wait()

      def grp(g, carry2):
        dv16 = dst_v[pl.ds(16 * g, 16)]
        drow_v[pl.ds(16 * g, 16)] = lax.shift_right_logical(dv16, 3)
        for j in range(16):
          k = 16 * g + j
          sv = ssb[k, pl.ds(0, 16)]
          dv = sdb[k, pl.ds(0, 16)]
          e = sv + dv
          e = jnp.maximum(e, 0.2 * e)
          ex = jnp.exp(e)
          goff = (dv16[j] & 7) * 16
          wd[k, pl.ds(goff, 16)] = ex
          for j2 in range(8):
            zj = zb[k, pl.ds(16 * j2, 16)]
            wb[k, pl.ds(16 * j2, 16)] = zj * ex
        return carry2

      lax.fori_loop(0, K // 16, grp, 0)
      pltpu.sync_copy(wb, u_sh.at[dst_v], add=True)
      pltpu.sync_copy(wd, d_sh.at[drow_v], add=True)

      def clean(g, carry2):
        dv16 = dst_v[pl.ds(16 * g, 16)]
        for j in range(16):
          goff = (dv16[j] & 7) * 16
          wd[16 * g + j, pl.ds(goff, 16)] = jnp.zeros((16,), jnp.float32)
        return carry2

      lax.fori_loop(0, K // 16, clean, 0)
      return carry

    lax.fori_loop(0, NCH, chunk, 0)
    plsc.subcore_barrier()
    pltpu.sync_copy(u_sh.at[pl.ds(s * RPT, RPT)],
                    u_out.at[c, pl.ds(s * RPT, RPT)])

    @pl.when(s == 0)
    def _():
      pltpu.sync_copy(u_sh.at[pl.ds(NS * RPT, TAIL)],
                      u_out.at[c, pl.ds(NS * RPT, TAIL)])
      pltpu.sync_copy(d_sh, d_out.at[c])

  return edge_pass


_edge_pass = _edge_pass_build()

_MB = 1112


def _proj_kernel(x_ref, w_ref, bs_ref, bd_ref, z_ref, sl_ref, sr_ref):
  z = jnp.dot(x_ref[:], w_ref[:], preferred_element_type=jnp.float32)
  z_ref[:] = z
  sl_ref[:] = jnp.dot(z, bs_ref[:], preferred_element_type=jnp.float32)
  sr_ref[:] = jnp.dot(z, bd_ref[:], preferred_element_type=jnp.float32)


def _proj(x, W, Bs, Bd):
  return pl.pallas_call(
      _proj_kernel,
      grid=(NP // _MB,),
      in_specs=[
          pl.BlockSpec((_MB, DIM), lambda i: (i, 0)),
          pl.BlockSpec((DIM, DIM), lambda i: (0, 0)),
          pl.BlockSpec((DIM, SW), lambda i: (0, 0)),
          pl.BlockSpec((DIM, SW), lambda i: (0, 0)),
      ],
      out_specs=[
          pl.BlockSpec((_MB, DIM), lambda i: (i, 0)),
          pl.BlockSpec((_MB, SW), lambda i: (i, 0)),
          pl.BlockSpec((_MB, SW), lambda i: (i, 0)),
      ],
      out_shape=[
          jax.ShapeDtypeStruct((NP, DIM), jnp.float32),
          jax.ShapeDtypeStruct((NP, SW), jnp.float32),
          jax.ShapeDtypeStruct((NP, SW), jnp.float32),
      ])(x, W, Bs, Bd)


def _combine1_kernel(u_ref, d_ref, p_ref, r_ref, w2_ref, bs_ref, bd_ref,
                     z2_ref, sl_ref, sr_ref):
  num = jnp.dot(u_ref[0] + u_ref[1], p_ref[:],
                preferred_element_type=jnp.float32)
  d = d_ref[0] + d_ref[1]
  den = jnp.dot(d[:, :8], r_ref[:], preferred_element_type=jnp.float32) + 1e-9
  h = num / den
  h = jnp.where(h > 0, h, jnp.exp(jnp.minimum(h, 0.0)) - 1.0)
  z2 = jnp.dot(h, w2_ref[:], preferred_element_type=jnp.float32)
  z2_ref[:] = z2
  sl_ref[:] = jnp.dot(z2, bs_ref[:], preferred_element_type=jnp.float32)
  sr_ref[:] = jnp.dot(z2, bd_ref[:], preferred_element_type=jnp.float32)


def _combine1(u, d, P, R, W2, Bs, Bd):
  return pl.pallas_call(
      _combine1_kernel,
      grid=(NP // _MB,),
      in_specs=[
          pl.BlockSpec((NC, _MB, DIM), lambda i: (0, i, 0)),
          pl.BlockSpec((NC, _MB, 16), lambda i: (0, i, 0)),
          pl.BlockSpec((DIM, DIM), lambda i: (0, 0)),
          pl.BlockSpec((8, DIM), lambda i: (0, 0)),
          pl.BlockSpec((DIM, DIM), lambda i: (0, 0)),
          pl.BlockSpec((DIM, SW), lambda i: (0, 0)),
          pl.BlockSpec((DIM, SW), lambda i: (0, 0)),
      ],
      out_specs=[
          pl.BlockSpec((_MB, DIM), lambda i: (i, 0)),
          pl.BlockSpec((_MB, SW), lambda i: (i, 0)),
          pl.BlockSpec((_MB, SW), lambda i: (i, 0)),
      ],
      out_shape=[
          jax.ShapeDtypeStruct((NP, DIM), jnp.float32),
          jax.ShapeDtypeStruct((NP, SW), jnp.float32),
          jax.ShapeDtypeStruct((NP, SW), jnp.float32),
      ])(u, d, P, R, W2, Bs, Bd)


def _combine2_kernel(u_ref, d_ref, r_ref, o_ref):
  num = u_ref[0] + u_ref[1]
  d = d_ref[0] + d_ref[1]
  den = jnp.dot(d[:, :8], r_ref[:], preferred_element_type=jnp.float32) + 1e-9
  o_ref[:] = num / den


def _combine2(u, d, R):
  return pl.pallas_call(
      _combine2_kernel,
      grid=(NP // _MB,),
      in_specs=[
          pl.BlockSpec((NC, _MB, DIM), lambda i: (0, i, 0)),
          pl.BlockSpec((NC, _MB, 16), lambda i: (0, i, 0)),
          pl.BlockSpec((8, DIM), lambda i: (0, 0)),
      ],
      out_specs=pl.BlockSpec((_MB, DIM), lambda i: (i, 0)),
      out_shape=jax.ShapeDtypeStruct((NP, DIM), jnp.float32))(u, d, R)


_R1 = np.kron(np.eye(8), np.ones((1, 16))).astype(np.float32)
_R2 = np.concatenate([np.ones((1, 128)), np.zeros((7, 128))]).astype(np.float32)


def kernel(x, edge_index, W1, a1_src, a1_dst, W2, a2_src, a2_dst):
  idx = jnp.arange(DIM)
  hh = idx // 16
  B1s = (jnp.zeros((DIM, SW), jnp.float32)
         .at[idx, hh].set(a1_src.reshape(-1))
         .at[idx, 8 + hh].set(a1_src.reshape(-1)))
  B1d = (jnp.zeros((DIM, SW), jnp.float32)
         .at[idx, hh].set(a1_dst.reshape(-1))
         .at[idx, 8 + hh].set(a1_dst.reshape(-1)))
  B2s = (jnp.zeros((DIM, SW), jnp.float32)
         .at[:, :16].set(jnp.broadcast_to(a2_src[0][:, None], (DIM, 16))))
  B2d = (jnp.zeros((DIM, SW), jnp.float32)
         .at[:, :16].set(jnp.broadcast_to(a2_dst[0][:, None], (DIM, 16))))
  W1t = W1[:, jnp.asarray(_PERM)]
  B1s = B1s[jnp.asarray(_PERM), :]
  B1d = B1d[jnp.asarray(_PERM), :]
  zeros_u = jnp.zeros((NP, DIM), jnp.float32)
  xp = jnp.zeros((NP, DIM), jnp.float32).at[:N].set(x)
  pad = jnp.full((NW, EPW - E // NW), N, jnp.int32)
  srcp = jnp.concatenate(
      [edge_index[0].reshape(NW, E // NW), pad], axis=1).reshape(-1)
  dstp = jnp.concatenate(
      [edge_index[1].reshape(NW, E // NW), pad], axis=1).reshape(-1)
  edges_flat = jnp.concatenate([srcp, dstp])

  zt1, sl1, sr1 = _proj(xp, W1t, B1s, B1d)
  u1, d1 = _edge_pass(zt1, sl1, sr1, edges_flat, zeros_u)
  z2, sl2, sr2 = _combine1(u1, d1.reshape(NC, DR * 8, 16)[:, :NP],
                           jnp.asarray(_PINV), _R1, W2, B2s, B2d)
  u2, d2 = _edge_pass(z2, sl2, sr2, edges_flat, zeros_u)
  return _combine2(u2, d2.reshape(NC, DR * 8, 16)[:, :NP], _R2)[:N]


# speedup vs baseline: 1.7711x; 1.7711x over previous
import functools

import jax
import jax.numpy as jnp
import numpy as np
from jax import lax
from jax.experimental import pallas as pl
from jax.experimental.pallas import tpu as pltpu
from jax.experimental.pallas import tpu_sc as plsc

N = 10000
E = 320000
DIM = 128
SW = 128
NC = 2
NS = 16
NW = NC * NS
NP = 10008
K = 32
NCH = 314
NG = NCH // 2
EPW = K * NCH
EP = NW * EPW
RPT = 624
TAIL = NP - NS * RPT
DR = 1256

_PERM = np.empty((DIM,), np.int64)
for _c in range(DIM):
  _j, _l = _c >> 4, _c & 15
  _PERM[_c] = (_l & 7) * 16 + 2 * _j + (_l >> 3)
_PINV = np.zeros((DIM, DIM), np.float32)
for _c in range(DIM):
  _PINV[_c, _PERM[_c]] = 1.0


def _edge_pass_build():
  mesh = plsc.VectorSubcoreMesh(core_axis_name="c", subcore_axis_name="s")

  @functools.partial(
      pl.kernel,
      out_type=(jax.ShapeDtypeStruct((NC, NP, DIM), jnp.float32),
                jax.ShapeDtypeStruct((NC, DR, DIM), jnp.float32)),
      mesh=mesh,
      scratch_types=[
          pltpu.VMEM((2, 64), jnp.int32),
          pltpu.VMEM((2, K), jnp.int32),
          pltpu.VMEM((2, K), jnp.int32),
          pltpu.VMEM((K,), jnp.int32),
          pltpu.VMEM((2, K, DIM), jnp.float32),
          pltpu.VMEM((2, K, SW), jnp.float32),
          pltpu.VMEM((2, K, SW), jnp.float32),
          pltpu.VMEM((K, DIM), jnp.float32),
          pltpu.VMEM((K, DIM), jnp.float32),
          pltpu.VMEM_SHARED((NP, DIM), jnp.float32),
          pltpu.VMEM_SHARED((DR, DIM), jnp.float32),
          pltpu.SemaphoreType.DMA,
          pltpu.SemaphoreType.DMA,
          pltpu.SemaphoreType.DMA,
          pltpu.SemaphoreType.DMA,
      ])
  def edge_pass(zt_hbm, sl_hbm, sr_hbm, edges_hbm, zeros_hbm, u_out, d_out,
                eb, src_v, dst_v, drow_v, zb, ssb, sdb, wb, wd, u_sh, d_sh,
                sg0, sg1, si0, si1):
    c = lax.axis_index("c")
    s = lax.axis_index("s")
    wid = s * NC + c
    sg = (sg0, sg1)
    si = (si0, si1)

    pltpu.sync_copy(zeros_hbm.at[pl.ds(s * RPT, RPT)],
                    u_sh.at[pl.ds(s * RPT, RPT)])

    @pl.when(s == 0)
    def _():
      pltpu.sync_copy(zeros_hbm.at[pl.ds(NS * RPT, TAIL)],
                      u_sh.at[pl.ds(NS * RPT, TAIL)])
      pltpu.sync_copy(zeros_hbm.at[pl.ds(0, DR)], d_sh)

    def zrow(k, carry):
      for j in range(8):
        wd[k, pl.ds(16 * j, 16)] = jnp.zeros((16,), jnp.float32)
      return carry

    lax.fori_loop(0, K, zrow, 0)
    plsc.subcore_barrier()

    cbase = wid * NCH

    def idx_fetch(ci, b):
      pltpu.async_copy(edges_hbm.at[pl.ds((cbase + ci) * 64, 64)],
                       eb.at[b], si[b])

    def prefetch(ci, b):
      pltpu.make_async_copy(edges_hbm.at[pl.ds(0, 64)], eb.at[b],
                            si[b]).wait()
      for t in range(K // 16):
        src_v[b, pl.ds(16 * t, 16)] = eb[b, pl.ds(16 * t, 16)]
        dst_v[b, pl.ds(16 * t, 16)] = eb[b, pl.ds(32 + 16 * t, 16)]
      pltpu.async_copy(zt_hbm.at[src_v.at[b]], zb.at[b], sg[b])
      pltpu.async_copy(sl_hbm.at[src_v.at[b]], ssb.at[b], sg[b])
      pltpu.async_copy(sr_hbm.at[dst_v.at[b]], sdb.at[b], sg[b])

      @pl.when(ci + 2 < NCH)
      def _():
        idx_fetch(ci + 2, b)

    def consume(b):
      pltpu.make_async_copy(zt_hbm.at[pl.ds(0, K)], zb.at[b], sg[b]).wait()
      pltpu.make_async_copy(sl_hbm.at[pl.ds(0, K)], ssb.at[b], sg[b]).wait()
      pltpu.make_async_copy(sr_hbm.at[pl.ds(0, K)], sdb.at[b], sg[b]).wait()

      def grp(g, carry2):
        dv16 = dst_v[b, pl.ds(16 * g, 16)]
        drow_v[pl.ds(16 * g, 16)] = lax.shift_right_logical(dv16, 3)
        for j in range(16):
          k = 16 * g + j
          sv = ssb[b, k, pl.ds(0, 16)]
          dv = sdb[b, k, pl.ds(0, 16)]
          e = sv + dv
          e = jnp.maximum(e, 0.2 * e)
          ex = jnp.exp(e)
          goff = (dv16[j] & 7) * 16
          wd[k, pl.ds(goff, 16)] = ex
          for j2 in range(8):
            zj = zb[b, k, pl.ds(16 * j2, 16)]
            wb[k, pl.ds(16 * j2, 16)] = zj * ex
        return carry2

      lax.fori_loop(0, K // 16, grp, 0)
      pltpu.sync_copy(wb, u_sh.at[dst_v.at[b]], add=True)
      pltpu.sync_copy(wd, d_sh.at[drow_v], add=True)

      def clean(g, carry2):
        dv16 = dst_v[b, pl.ds(16 * g, 16)]
        for j in range(16):
          goff = (dv16[j] & 7) * 16
          wd[16 * g + j, pl.ds(goff, 16)] = jnp.zeros((16,), jnp.float32)
        return carry2

      lax.fori_loop(0, K // 16, clean, 0)

    idx_fetch(0, 0)
    idx_fetch(1, 1)
    prefetch(0, 0)

    def step(g, carry):
      ci0 = 2 * g
      prefetch(ci0 + 1, 1)
      consume(0)

      @pl.when(g < NG - 1)
      def _():
        prefetch(ci0 + 2, 0)

      consume(1)
      return carry

    lax.fori_loop(0, NG, step, 0)
    plsc.subcore_barrier()
    pltpu.sync_copy(u_sh.at[pl.ds(s * RPT, RPT)],
                    u_out.at[c, pl.ds(s * RPT, RPT)])

    @pl.when(s == 0)
    def _():
      pltpu.sync_copy(u_sh.at[pl.ds(NS * RPT, TAIL)],
                      u_out.at[c, pl.ds(NS * RPT, TAIL)])
      pltpu.sync_copy(d_sh, d_out.at[c])

  return edge_pass


_edge_pass = _edge_pass_build()

_MB = 1112


def _proj_kernel(x_ref, w_ref, bs_ref, bd_ref, z_ref, sl_ref, sr_ref):
  z = jnp.dot(x_ref[:], w_ref[:], preferred_element_type=jnp.float32)
  z_ref[:] = z
  sl_ref[:] = jnp.dot(z, bs_ref[:], preferred_element_type=jnp.float32)
  sr_ref[:] = jnp.dot(z, bd_ref[:], preferred_element_type=jnp.float32)


def _proj(x, W, Bs, Bd):
  return pl.pallas_call(
      _proj_kernel,
      grid=(NP // _MB,),
      in_specs=[
          pl.BlockSpec((_MB, DIM), lambda i: (i, 0)),
          pl.BlockSpec((DIM, DIM), lambda i: (0, 0)),
          pl.BlockSpec((DIM, SW), lambda i: (0, 0)),
          pl.BlockSpec((DIM, SW), lambda i: (0, 0)),
      ],
      out_specs=[
          pl.BlockSpec((_MB, DIM), lambda i: (i, 0)),
          pl.BlockSpec((_MB, SW), lambda i: (i, 0)),
          pl.BlockSpec((_MB, SW), lambda i: (i, 0)),
      ],
      out_shape=[
          jax.ShapeDtypeStruct((NP, DIM), jnp.float32),
          jax.ShapeDtypeStruct((NP, SW), jnp.float32),
          jax.ShapeDtypeStruct((NP, SW), jnp.float32),
      ])(x, W, Bs, Bd)


def _combine1_kernel(u_ref, d_ref, p_ref, r_ref, w2_ref, bs_ref, bd_ref,
                     z2_ref, sl_ref, sr_ref):
  num = jnp.dot(u_ref[0] + u_ref[1], p_ref[:],
                preferred_element_type=jnp.float32)
  d = d_ref[0] + d_ref[1]
  den = jnp.dot(d[:, :8], r_ref[:], preferred_element_type=jnp.float32) + 1e-9
  h = num / den
  h = jnp.where(h > 0, h, jnp.exp(jnp.minimum(h, 0.0)) - 1.0)
  z2 = jnp.dot(h, w2_ref[:], preferred_element_type=jnp.float32)
  z2_ref[:] = z2
  sl_ref[:] = jnp.dot(z2, bs_ref[:], preferred_element_type=jnp.float32)
  sr_ref[:] = jnp.dot(z2, bd_ref[:], preferred_element_type=jnp.float32)


def _combine1(u, d, P, R, W2, Bs, Bd):
  return pl.pallas_call(
      _combine1_kernel,
      grid=(NP // _MB,),
      in_specs=[
          pl.BlockSpec((NC, _MB, DIM), lambda i: (0, i, 0)),
          pl.BlockSpec((NC, _MB, 16), lambda i: (0, i, 0)),
          pl.BlockSpec((DIM, DIM), lambda i: (0, 0)),
          pl.BlockSpec((8, DIM), lambda i: (0, 0)),
          pl.BlockSpec((DIM, DIM), lambda i: (0, 0)),
          pl.BlockSpec((DIM, SW), lambda i: (0, 0)),
          pl.BlockSpec((DIM, SW), lambda i: (0, 0)),
      ],
      out_specs=[
          pl.BlockSpec((_MB, DIM), lambda i: (i, 0)),
          pl.BlockSpec((_MB, SW), lambda i: (i, 0)),
          pl.BlockSpec((_MB, SW), lambda i: (i, 0)),
      ],
      out_shape=[
          jax.ShapeDtypeStruct((NP, DIM), jnp.float32),
          jax.ShapeDtypeStruct((NP, SW), jnp.float32),
          jax.ShapeDtypeStruct((NP, SW), jnp.float32),
      ])(u, d, P, R, W2, Bs, Bd)


def _combine2_kernel(u_ref, d_ref, r_ref, o_ref):
  num = u_ref[0] + u_ref[1]
  d = d_ref[0] + d_ref[1]
  den = jnp.dot(d[:, :8], r_ref[:], preferred_element_type=jnp.float32) + 1e-9
  o_ref[:] = num / den


def _combine2(u, d, R):
  return pl.pallas_call(
      _combine2_kernel,
      grid=(NP // _MB,),
      in_specs=[
          pl.BlockSpec((NC, _MB, DIM), lambda i: (0, i, 0)),
          pl.BlockSpec((NC, _MB, 16), lambda i: (0, i, 0)),
          pl.BlockSpec((8, DIM), lambda i: (0, 0)),
      ],
      out_specs=pl.BlockSpec((_MB, DIM), lambda i: (i, 0)),
      out_shape=jax.ShapeDtypeStruct((NP, DIM), jnp.float32))(u, d, R)


_R1 = np.kron(np.eye(8), np.ones((1, 16))).astype(np.float32)
_R2 = np.concatenate([np.ones((1, 128)), np.zeros((7, 128))]).astype(np.float32)


def kernel(x, edge_index, W1, a1_src, a1_dst, W2, a2_src, a2_dst):
  idx = jnp.arange(DIM)
  hh = idx // 16
  B1s = (jnp.zeros((DIM, SW), jnp.float32)
         .at[idx, hh].set(a1_src.reshape(-1))
         .at[idx, 8 + hh].set(a1_src.reshape(-1)))
  B1d = (jnp.zeros((DIM, SW), jnp.float32)
         .at[idx, hh].set(a1_dst.reshape(-1))
         .at[idx, 8 + hh].set(a1_dst.reshape(-1)))
  B2s = (jnp.zeros((DIM, SW), jnp.float32)
         .at[:, :16].set(jnp.broadcast_to(a2_src[0][:, None], (DIM, 16))))
  B2d = (jnp.zeros((DIM, SW), jnp.float32)
         .at[:, :16].set(jnp.broadcast_to(a2_dst[0][:, None], (DIM, 16))))
  W1t = W1[:, jnp.asarray(_PERM)]
  B1s = B1s[jnp.asarray(_PERM), :]
  B1d = B1d[jnp.asarray(_PERM), :]
  zeros_u = jnp.zeros((NP, DIM), jnp.float32)
  xp = jnp.zeros((NP, DIM), jnp.float32).at[:N].set(x)
  pad = jnp.full((NW, EPW - E // NW), N, jnp.int32)
  srcp = jnp.concatenate(
      [edge_index[0].reshape(NW, E // NW), pad], axis=1).reshape(NW, NCH, K)
  dstp = jnp.concatenate(
      [edge_index[1].reshape(NW, E // NW), pad], axis=1).reshape(NW, NCH, K)
  edges_flat = jnp.concatenate([srcp, dstp], axis=2).reshape(-1)

  zt1, sl1, sr1 = _proj(xp, W1t, B1s, B1d)
  u1, d1 = _edge_pass(zt1, sl1, sr1, edges_flat, zeros_u)
  z2, sl2, sr2 = _combine1(u1, d1.reshape(NC, DR * 8, 16)[:, :NP],
                           jnp.asarray(_PINV), _R1, W2, B2s, B2d)
  u2, d2 = _edge_pass(z2, sl2, sr2, edges_flat, zeros_u)
  return _combine2(u2, d2.reshape(NC, DR * 8, 16)[:, :NP], _R2)[:N]


# speedup vs baseline: 1.7908x; 1.0111x over previous
import functools

import jax
import jax.numpy as jnp
import numpy as np
from jax import lax
from jax.experimental import pallas as pl
from jax.experimental.pallas import tpu as pltpu
from jax.experimental.pallas import tpu_sc as plsc

N = 10000
E = 320000
DIM = 128
SW = 128
NC = 2
NS = 16
NW = NC * NS
NP = 10008
K = 32
NCH = 314
NG = NCH // 2
EPW = K * NCH
EP = NW * EPW
DR = 1256
NPX = NP + DR
RPT = NPX // NS

_PERM = np.empty((DIM,), np.int64)
for _c in range(DIM):
  _j, _l = _c >> 4, _c & 15
  _PERM[_c] = (_l & 7) * 16 + 2 * _j + (_l >> 3)
_PINV = np.zeros((DIM, DIM), np.float32)
for _c in range(DIM):
  _PINV[_c, _PERM[_c]] = 1.0


def _edge_pass_build():
  mesh = plsc.VectorSubcoreMesh(core_axis_name="c", subcore_axis_name="s")

  @functools.partial(
      pl.kernel,
      out_type=jax.ShapeDtypeStruct((NC, NPX, DIM), jnp.float32),
      mesh=mesh,
      scratch_types=[
          pltpu.VMEM((2, K), jnp.int32),
          pltpu.VMEM((2, 2 * K), jnp.int32),
          pltpu.VMEM((2, K, DIM), jnp.float32),
          pltpu.VMEM((2, K, SW), jnp.float32),
          pltpu.VMEM((2, K, SW), jnp.float32),
          pltpu.VMEM((2 * K, DIM), jnp.float32),
          pltpu.VMEM_SHARED((NPX, DIM), jnp.float32),
          pltpu.SemaphoreType.DMA,
          pltpu.SemaphoreType.DMA,
          pltpu.SemaphoreType.DMA,
          pltpu.SemaphoreType.DMA,
      ])
  def edge_pass(zt_hbm, sl_hbm, sr_hbm, edges_hbm, zeros_hbm, u_out,
                src_v, scidx, zb, ssb, sdb, wbd, u_sh,
                sg0, sg1, si0, si1):
    c = lax.axis_index("c")
    s = lax.axis_index("s")
    wid = s * NC + c
    sg = (sg0, sg1)
    si = (si0, si1)

    pltpu.sync_copy(zeros_hbm.at[pl.ds(s * RPT, RPT)],
                    u_sh.at[pl.ds(s * RPT, RPT)])

    def zrow(k, carry):
      for j in range(8):
        wbd[K + k, pl.ds(16 * j, 16)] = jnp.zeros((16,), jnp.float32)
      return carry

    lax.fori_loop(0, K, zrow, 0)
    plsc.subcore_barrier()

    cbase = wid * NCH

    def idx_fetch(ci, b):
      off = (cbase + ci) * (2 * K)
      pltpu.async_copy(edges_hbm.at[pl.ds(off, K)], src_v.at[b], si[b])
      pltpu.async_copy(edges_hbm.at[pl.ds(off + K, K)],
                       scidx.at[b, pl.ds(0, K)], si[b])

    def prefetch(ci, b):
      pltpu.make_async_copy(edges_hbm.at[pl.ds(0, K)], src_v.at[b],
                            si[b]).wait()
      pltpu.make_async_copy(edges_hbm.at[pl.ds(0, K)],
                            scidx.at[b, pl.ds(0, K)], si[b]).wait()
      pltpu.async_copy(zt_hbm.at[src_v.at[b]], zb.at[b], sg[b])
      pltpu.async_copy(sl_hbm.at[src_v.at[b]], ssb.at[b], sg[b])
      pltpu.async_copy(sr_hbm.at[scidx.at[b, pl.ds(0, K)]], sdb.at[b], sg[b])

      @pl.when(ci + 2 < NCH)
      def _():
        idx_fetch(ci + 2, b)

    def consume(b):
      pltpu.make_async_copy(zt_hbm.at[pl.ds(0, K)], zb.at[b], sg[b]).wait()
      pltpu.make_async_copy(sl_hbm.at[pl.ds(0, K)], ssb.at[b], sg[b]).wait()
      pltpu.make_async_copy(sr_hbm.at[pl.ds(0, K)], sdb.at[b], sg[b]).wait()

      def grp(g, carry2):
        dv16 = scidx[b, pl.ds(16 * g, 16)]
        scidx[b, pl.ds(K + 16 * g, 16)] = (
            lax.shift_right_logical(dv16, 3) + NP)
        for j in range(16):
          k = 16 * g + j
          sv = ssb[b, k, pl.ds(0, 16)]
          dv = sdb[b, k, pl.ds(0, 16)]
          e = sv + dv
          e = jnp.maximum(e, 0.2 * e)
          ex = jnp.exp(e)
          goff = (dv16[j] & 7) * 16
          wbd[K + k, pl.ds(goff, 16)] = ex
          for j2 in range(8):
            zj = zb[b, k, pl.ds(16 * j2, 16)]
            wbd[k, pl.ds(16 * j2, 16)] = zj * ex
        return carry2

      lax.fori_loop(0, K // 16, grp, 0)
      pltpu.sync_copy(wbd, u_sh.at[scidx.at[b]], add=True)

      def clean(g, carry2):
        dv16 = scidx[b, pl.ds(16 * g, 16)]
        for j in range(16):
          goff = (dv16[j] & 7) * 16
          wbd[K + 16 * g + j, pl.ds(goff, 16)] = jnp.zeros((16,), jnp.float32)
        return carry2

      lax.fori_loop(0, K // 16, clean, 0)

    idx_fetch(0, 0)
    idx_fetch(1, 1)
    prefetch(0, 0)

    def step(g, carry):
      ci0 = 2 * g
      prefetch(ci0 + 1, 1)
      consume(0)

      @pl.when(g < NG - 1)
      def _():
        prefetch(ci0 + 2, 0)

      consume(1)
      return carry

    lax.fori_loop(0, NG, step, 0)
    plsc.subcore_barrier()
    pltpu.sync_copy(u_sh.at[pl.ds(s * RPT, RPT)],
                    u_out.at[c, pl.ds(s * RPT, RPT)])

  return edge_pass


_edge_pass = _edge_pass_build()

_MB = 1112


def _proj_kernel(x_ref, w_ref, bs_ref, bd_ref, z_ref, sl_ref, sr_ref):
  z = jnp.dot(x_ref[:], w_ref[:], preferred_element_type=jnp.float32)
  z_ref[:] = z
  sl_ref[:] = jnp.dot(z, bs_ref[:], preferred_element_type=jnp.float32)
  sr_ref[:] = jnp.dot(z, bd_ref[:], preferred_element_type=jnp.float32)


def _proj(x, W, Bs, Bd):
  return pl.pallas_call(
      _proj_kernel,
      grid=(NP // _MB,),
      in_specs=[
          pl.BlockSpec((_MB, DIM), lambda i: (i, 0)),
          pl.BlockSpec((DIM, DIM), lambda i: (0, 0)),
          pl.BlockSpec((DIM, SW), lambda i: (0, 0)),
          pl.BlockSpec((DIM, SW), lambda i: (0, 0)),
      ],
      out_specs=[
          pl.BlockSpec((_MB, DIM), lambda i: (i, 0)),
          pl.BlockSpec((_MB, SW), lambda i: (i, 0)),
          pl.BlockSpec((_MB, SW), lambda i: (i, 0)),
      ],
      out_shape=[
          jax.ShapeDtypeStruct((NP, DIM), jnp.float32),
          jax.ShapeDtypeStruct((NP, SW), jnp.float32),
          jax.ShapeDtypeStruct((NP, SW), jnp.float32),
      ])(x, W, Bs, Bd)


def _combine1_kernel(u_ref, d_ref, p_ref, r_ref, w2_ref, bs_ref, bd_ref,
                     z2_ref, sl_ref, sr_ref):
  num = jnp.dot(u_ref[0] + u_ref[1], p_ref[:],
                preferred_element_type=jnp.float32)
  d = d_ref[0] + d_ref[1]
  den = jnp.dot(d[:, :8], r_ref[:], preferred_element_type=jnp.float32) + 1e-9
  h = num / den
  h = jnp.where(h > 0, h, jnp.exp(jnp.minimum(h, 0.0)) - 1.0)
  z2 = jnp.dot(h, w2_ref[:], preferred_element_type=jnp.float32)
  z2_ref[:] = z2
  sl_ref[:] = jnp.dot(z2, bs_ref[:], preferred_element_type=jnp.float32)
  sr_ref[:] = jnp.dot(z2, bd_ref[:], preferred_element_type=jnp.float32)


def _combine1(u, d, P, R, W2, Bs, Bd):
  return pl.pallas_call(
      _combine1_kernel,
      grid=(NP // _MB,),
      in_specs=[
          pl.BlockSpec((NC, _MB, DIM), lambda i: (0, i, 0)),
          pl.BlockSpec((NC, _MB, 16), lambda i: (0, i, 0)),
          pl.BlockSpec((DIM, DIM), lambda i: (0, 0)),
          pl.BlockSpec((8, DIM), lambda i: (0, 0)),
          pl.BlockSpec((DIM, DIM), lambda i: (0, 0)),
          pl.BlockSpec((DIM, SW), lambda i: (0, 0)),
          pl.BlockSpec((DIM, SW), lambda i: (0, 0)),
      ],
      out_specs=[
          pl.BlockSpec((_MB, DIM), lambda i: (i, 0)),
          pl.BlockSpec((_MB, SW), lambda i: (i, 0)),
          pl.BlockSpec((_MB, SW), lambda i: (i, 0)),
      ],
      out_shape=[
          jax.ShapeDtypeStruct((NP, DIM), jnp.float32),
          jax.ShapeDtypeStruct((NP, SW), jnp.float32),
          jax.ShapeDtypeStruct((NP, SW), jnp.float32),
      ])(u, d, P, R, W2, Bs, Bd)


def _combine2_kernel(u_ref, d_ref, r_ref, o_ref):
  num = u_ref[0] + u_ref[1]
  d = d_ref[0] + d_ref[1]
  den = jnp.dot(d[:, :8], r_ref[:], preferred_element_type=jnp.float32) + 1e-9
  o_ref[:] = num / den


def _combine2(u, d, R):
  return pl.pallas_call(
      _combine2_kernel,
      grid=(NP // _MB,),
      in_specs=[
          pl.BlockSpec((NC, _MB, DIM), lambda i: (0, i, 0)),
          pl.BlockSpec((NC, _MB, 16), lambda i: (0, i, 0)),
          pl.BlockSpec((8, DIM), lambda i: (0, 0)),
      ],
      out_specs=pl.BlockSpec((_MB, DIM), lambda i: (i, 0)),
      out_shape=jax.ShapeDtypeStruct((NP, DIM), jnp.float32))(u, d, R)


_R1 = np.kron(np.eye(8), np.ones((1, 16))).astype(np.float32)
_R2 = np.concatenate([np.ones((1, 128)), np.zeros((7, 128))]).astype(np.float32)


def kernel(x, edge_index, W1, a1_src, a1_dst, W2, a2_src, a2_dst):
  idx = jnp.arange(DIM)
  hh = idx // 16
  B1s = (jnp.zeros((DIM, SW), jnp.float32)
         .at[idx, hh].set(a1_src.reshape(-1))
         .at[idx, 8 + hh].set(a1_src.reshape(-1)))
  B1d = (jnp.zeros((DIM, SW), jnp.float32)
         .at[idx, hh].set(a1_dst.reshape(-1))
         .at[idx, 8 + hh].set(a1_dst.reshape(-1)))
  B2s = (jnp.zeros((DIM, SW), jnp.float32)
         .at[:, :16].set(jnp.broadcast_to(a2_src[0][:, None], (DIM, 16))))
  B2d = (jnp.zeros((DIM, SW), jnp.float32)
         .at[:, :16].set(jnp.broadcast_to(a2_dst[0][:, None], (DIM, 16))))
  W1t = W1[:, jnp.asarray(_PERM)]
  B1s = B1s[jnp.asarray(_PERM), :]
  B1d = B1d[jnp.asarray(_PERM), :]
  zeros_u = jnp.zeros((NPX, DIM), jnp.float32)
  xp = jnp.zeros((NP, DIM), jnp.float32).at[:N].set(x)
  pad = jnp.full((NW, EPW - E // NW), N, jnp.int32)
  srcp = jnp.concatenate(
      [edge_index[0].reshape(NW, E // NW), pad], axis=1).reshape(NW, NCH, K)
  dstp = jnp.concatenate(
      [edge_index[1].reshape(NW, E // NW), pad], axis=1).reshape(NW, NCH, K)
  edges_flat = jnp.concatenate([srcp, dstp], axis=2).reshape(-1)

  zt1, sl1, sr1 = _proj(xp, W1t, B1s, B1d)
  ud1 = _edge_pass(zt1, sl1, sr1, edges_flat, zeros_u)
  u1 = ud1[:, :NP]
  d1 = ud1[:, NP:].reshape(NC, DR * 8, 16)[:, :NP]
  z2, sl2, sr2 = _combine1(u1, d1, jnp.asarray(_PINV), _R1, W2, B2s, B2d)
  ud2 = _edge_pass(z2, sl2, sr2, edges_flat, zeros_u)
  u2 = ud2[:, :NP]
  d2 = ud2[:, NP:].reshape(NC, DR * 8, 16)[:, :NP]
  return _combine2(u2, d2, _R2)[:N]
